# baseline (speedup 1.0000x reference)
import functools

import jax
import jax.numpy as jnp
from jax import lax
from jax.experimental import pallas as pl
from jax.experimental.pallas import tpu as pltpu
from jax.experimental.pallas import tpu_sc as plsc

_D = 1024
_M = 64
_CORE = 256
_H = 4
_HD = 64
_N = 32768
_BN = 2048

_NW = 32
_TPB = _N // _NW
_CH = 256

_NT = (((1,), (1,)), ((), ()))
_NN = (((1,), (0,)), ((), ()))


def _a_body(x_ref, gm_ref, wq_ref, bq_ref, p1_ref, p1b_ref, p2_ref, p2b_ref,
            mb_ref, s_out_ref, s2_out_ref, kt_out_ref, q_scr):
    i = pl.program_id(0)

    @pl.when(i == 0)
    def _init():
        gm = gm_ref[...]
        r1 = jax.nn.relu(
            jax.lax.dot_general(gm, p1_ref[...], _NT,
                                preferred_element_type=jnp.float32)
            + p1b_ref[...])
        q = jax.lax.dot_general(r1, p2_ref[...], _NT,
                                preferred_element_type=jnp.float32) \
            + p2b_ref[...]
        qt = jnp.concatenate([q, q, q, q], axis=0)
        r_i = jax.lax.broadcasted_iota(jnp.int32, (_CORE, _CORE), 0) // _HD
        c_i = jax.lax.broadcasted_iota(jnp.int32, (_CORE, _CORE), 1) // _HD
        q_scr[...] = jnp.where(r_i == c_i, qt * 0.125, 0.0)

    x = x_ref[...]
    kt = jax.lax.dot_general(wq_ref[...], x, _NT,
                             preferred_element_type=jnp.float32) + bq_ref[...]
    s = jax.lax.dot_general(q_scr[...], kt, _NN,
                            preferred_element_type=jnp.float32) \
        + mb_ref[...] * 5.0
    s_out_ref[...] = s
    s2_out_ref[...] = s
    kt_out_ref[...] = kt


def _const(shape):
    return pl.BlockSpec(shape, lambda i: tuple(0 for _ in shape))


@jax.jit
def _stage_a(flat, gm, wq, bq_c, p1, p1b_r, p2, p2b_r, mb_c):
    nb = _N // _BN
    return pl.pallas_call(
        _a_body,
        grid=(nb,),
        in_specs=[
            pl.BlockSpec((_BN, _D), lambda i: (i, 0)),
            _const((_M, _CORE)),
            _const((_CORE, _D)),
            _const((_CORE, 1)),
            _const((_D, _CORE)),
            _const((1, _D)),
            _const((_CORE, _D)),
            _const((1, _CORE)),
            _const((_CORE, 1)),
        ],
        out_specs=[
            pl.BlockSpec((_CORE, _BN), lambda i: (0, i)),
            pl.BlockSpec((_CORE, _BN), lambda i: (0, i)),
            pl.BlockSpec((_CORE, _BN), lambda i: (0, i)),
        ],
        out_shape=[
            jax.ShapeDtypeStruct((_CORE, _N), jnp.float32),
            jax.ShapeDtypeStruct((_CORE, _N), jnp.float32),
            jax.ShapeDtypeStruct((_CORE, _N), jnp.float32),
        ],
        scratch_shapes=[pltpu.VMEM((_CORE, _CORE), jnp.float32)],
        compiler_params=pltpu.CompilerParams(
            dimension_semantics=("arbitrary",)),
    )(flat, gm, wq, bq_c, p1, p1b_r, p2, p2b_r, mb_c)


def _sc_body(s_hbm, idx_hbm, s_v, idx_v, sem):
    wid = lax.axis_index("s") * 2 + lax.axis_index("c")
    base = wid * _TPB
    lane = lax.iota(jnp.int32, 16)

    def chunk(ci, _):
        cbase = base + ci * _CH
        pltpu.sync_copy(s_hbm.at[:, pl.ds(cbase, _CH)], s_v)

        def group(g, _):
            goff = g * 16

            def head(h, _):
                neg = jnp.full((16,), -3e38, jnp.float32)
                zero = jnp.zeros((16,), jnp.int32)

                def row(r, carry):
                    m1, i1, m2, i2 = carry
                    v = s_v[h * _M + r, pl.ds(goff, 16)]
                    rv = jnp.full((16,), r, jnp.int32)
                    gt1 = v > m1
                    gt2 = v > m2
                    m2n = jnp.where(gt1, m1, jnp.where(gt2, v, m2))
                    i2n = jnp.where(gt1, i1, jnp.where(gt2, rv, i2))
                    m1n = jnp.where(gt1, v, m1)
                    i1n = jnp.where(gt1, rv, i1)
                    return (m1n, i1n, m2n, i2n)

                m1, i1, m2, i2 = lax.fori_loop(
                    0, _M, row, (neg, zero, neg, zero))
                idx_v[2 * h, pl.ds(goff, 16)] = i1
                idx_v[2 * h + 1, pl.ds(goff, 16)] = i2
                return 0

            return lax.fori_loop(0, _H, head, 0)

        lax.fori_loop(0, _CH // 16, group, 0)
        pltpu.sync_copy(idx_v, idx_hbm.at[:, pl.ds(cbase, _CH)])
        return 0

    lax.fori_loop(0, _TPB // _CH, chunk, 0)


@jax.jit
def _stage_b(s_all):
    mesh = plsc.VectorSubcoreMesh(core_axis_name="c", subcore_axis_name="s")
    f = functools.partial(
        pl.kernel,
        mesh=mesh,
        out_type=jax.ShapeDtypeStruct((8, _N), jnp.int32),
        scratch_types=[
            pltpu.VMEM((_CORE, _CH), jnp.float32),
            pltpu.VMEM((8, _CH), jnp.int32),
            pltpu.SemaphoreType.DMA,
        ],
    )(_sc_body)
    return f(s_all)


def _c_body(kt_ref, idx_ref, gm_ref, lnw_ref, lnb_ref, u1_ref, u1b_ref,
            u2_ref, u2b_ref, now_ref, nob_ref,
            dmc_ref, lf_ref, acc_scr, lf_scr):
    i = pl.program_id(0)
    nb = pl.num_programs(0)

    @pl.when(i == 0)
    def _init():
        acc_scr[...] = jnp.zeros_like(acc_scr)
        lf_scr[...] = jnp.zeros_like(lf_scr)

    kt = kt_ref[...]
    iota = jax.lax.broadcasted_iota(jnp.int32, (_M, _BN), 0)
    ones_row = jnp.ones((1, _BN), jnp.float32)
    for h in range(_H):
        i1 = idx_ref[2 * h:2 * h + 1, :]
        i2 = idx_ref[2 * h + 1:2 * h + 2, :]
        gating = ((iota == i1) | (iota == i2)).astype(jnp.float32)
        acc_scr[h * _M:(h + 1) * _M, :] += jax.lax.dot_general(
            gating, kt[h * _HD:(h + 1) * _HD, :], _NT,
            preferred_element_type=jnp.float32)
        lf_scr[...] += jax.lax.dot_general(
            gating, ones_row, _NT,
            preferred_element_type=jnp.float32)

    @pl.when(i == nb - 1)
    def _fin():
        att = jnp.concatenate(
            [acc_scr[h * _M:(h + 1) * _M, :] for h in range(_H)],
            axis=1)
        ui = jnp.concatenate([gm_ref[...], att], axis=1)
        mu = jnp.mean(ui, axis=1, keepdims=True)
        var = jnp.mean((ui - mu) ** 2, axis=1, keepdims=True)
        xn = (ui - mu) * jax.lax.rsqrt(var + 1e-5) * lnw_ref[...] + lnb_ref[...]
        h1 = jax.nn.relu(
            jax.lax.dot_general(xn, u1_ref[...], _NT,
                                preferred_element_type=jnp.float32)
            + u1b_ref[...])
        h2 = jax.lax.dot_general(h1, u2_ref[...], _NT,
                                 preferred_element_type=jnp.float32) \
            + u2b_ref[...]
        mu2 = jnp.mean(h2, axis=1, keepdims=True)
        var2 = jnp.mean((h2 - mu2) ** 2, axis=1, keepdims=True)
        dmc_ref[...] = (h2 - mu2) * jax.lax.rsqrt(var2 + 1e-5) \
            * now_ref[...] + nob_ref[...]
        lf_ref[...] = lf_scr[...] * (1.0 / _H)


@jax.jit
def _stage_c(kt_all, idx_all, gm, lnw_r, lnb_r, u1, u1b_r, u2, u2b_r,
             now_r, nob_r):
    nb = _N // _BN
    return pl.pallas_call(
        _c_body,
        grid=(nb,),
        in_specs=[
            pl.BlockSpec((_CORE, _BN), lambda i: (0, i)),
            pl.BlockSpec((8, _BN), lambda i: (0, i)),
            _const((_M, _CORE)),
            _const((1, 2 * _CORE)),
            _const((1, 2 * _CORE)),
            _const((2 * _CORE, 2 * _CORE)),
            _const((1, 2 * _CORE)),
            _const((_CORE, 2 * _CORE)),
            _const((1, _CORE)),
            _const((1, _CORE)),
            _const((1, _CORE)),
        ],
        out_specs=[
            _const((_M, _CORE)),
            _const((_M, 1)),
        ],
        out_shape=[
            jax.ShapeDtypeStruct((_M, _CORE), jnp.float32),
            jax.ShapeDtypeStruct((_M, 1), jnp.float32),
        ],
        scratch_shapes=[
            pltpu.VMEM((_CORE, _HD), jnp.float32),
            pltpu.VMEM((_M, 1), jnp.float32),
        ],
        compiler_params=pltpu.CompilerParams(
            dimension_semantics=("arbitrary",)),
    )(kt_all, idx_all, gm, lnw_r, lnb_r, u1, u1b_r, u2, u2b_r, now_r, nob_r)


def kernel(batch_queries, global_memory_base, Wq, bq, P1, p1b, P2, p2b,
           mem_bias, ln_w, ln_b, U1, U1b, U2, U2b, no_w, no_b):
    flat = batch_queries.reshape(_N, _D)
    gm = global_memory_base.reshape(_M, _CORE)
    s_all, s2_all, kt_all = _stage_a(
        flat, gm, Wq, bq.reshape(_CORE, 1), P1, p1b.reshape(1, _D),
        P2, p2b.reshape(1, _CORE), mem_bias.reshape(_CORE, 1))
    idx_all = _stage_b(s_all)
    dmc, lf = _stage_c(
        kt_all, idx_all, gm, ln_w.reshape(1, 2 * _CORE),
        ln_b.reshape(1, 2 * _CORE), U1, U1b.reshape(1, 2 * _CORE),
        U2, U2b.reshape(1, _CORE), no_w.reshape(1, _CORE),
        no_b.reshape(1, _CORE))
    return (dmc.reshape(1, _M, _CORE), s_all.reshape(_H, _M, _N),
            s2_all.reshape(_H, _M, _N), lf.reshape(_M))


# speedup vs baseline: 2.6169x; 2.6169x over previous
import functools

import jax
import jax.numpy as jnp
from jax.experimental import pallas as pl
from jax.experimental.pallas import tpu as pltpu

_D = 1024
_M = 64
_CORE = 256
_H = 4
_HD = 64
_N = 32768
_BN = 4096

_NT = (((1,), (1,)), ((), ()))
_NN = (((1,), (0,)), ((), ()))


def _top2_gating(sub):
    m1 = jnp.max(sub, axis=0, keepdims=True)
    m2 = jnp.max(jnp.where(sub == m1, -3e38, sub), axis=0, keepdims=True)
    return (sub >= m2).astype(jnp.bfloat16)


def _body(x_ref, gm_ref, wq_ref, bq_ref, p1_ref, p1b_ref, p2_ref, p2b_ref,
          mb_ref, lnw_ref, lnb_ref, u1_ref, u1b_ref, u2_ref, u2b_ref,
          now_ref, nob_ref,
          s_out_ref, s2_out_ref, dmc_ref, lf_ref,
          q_scr, acc_scr):
    i = pl.program_id(0)
    nb = pl.num_programs(0)

    @pl.when(i == 0)
    def _init():
        gm = gm_ref[...]
        r1 = jax.nn.relu(
            jax.lax.dot_general(gm, p1_ref[...], _NT,
                                preferred_element_type=jnp.float32)
            + p1b_ref[...])
        q = jax.lax.dot_general(r1, p2_ref[...], _NT,
                                preferred_element_type=jnp.float32) \
            + p2b_ref[...]
        qt = jnp.concatenate([q, q, q, q], axis=0)
        r_i = jax.lax.broadcasted_iota(jnp.int32, (_CORE, _CORE), 0) // _HD
        c_i = jax.lax.broadcasted_iota(jnp.int32, (_CORE, _CORE), 1) // _HD
        q_scr[...] = jnp.where(r_i == c_i, qt * 0.125, 0.0)
        acc_scr[...] = jnp.zeros_like(acc_scr)

    x = x_ref[...]
    kt = jax.lax.dot_general(wq_ref[...], x, _NT,
                             preferred_element_type=jnp.float32) \
        + bq_ref[...]
    s = jax.lax.dot_general(q_scr[...], kt, _NN,
                            preferred_element_type=jnp.float32) \
        + mb_ref[...] * 5.0
    s_out_ref[...] = s
    s2_out_ref[...] = s

    ones_rows = jnp.ones((8, _BN), jnp.bfloat16)
    kt16 = kt.astype(jnp.bfloat16)
    for h in range(_H):
        sub = s[h * _M:(h + 1) * _M, :]
        gating = _top2_gating(sub)
        rhs = jnp.concatenate(
            [kt16[h * _HD:(h + 1) * _HD, :], ones_rows], axis=0)
        acc_scr[h * _M:(h + 1) * _M, :] += jax.lax.dot_general(
            gating, rhs, _NT,
            preferred_element_type=jnp.float32)

    @pl.when(i == nb - 1)
    def _fin():
        att = jnp.concatenate(
            [acc_scr[h * _M:(h + 1) * _M, 0:_HD] for h in range(_H)],
            axis=1)
        ui = jnp.concatenate([gm_ref[...], att], axis=1)
        mu = jnp.mean(ui, axis=1, keepdims=True)
        var = jnp.mean((ui - mu) ** 2, axis=1, keepdims=True)
        xn = (ui - mu) * jax.lax.rsqrt(var + 1e-5) * lnw_ref[...] + lnb_ref[...]
        h1 = jax.nn.relu(
            jax.lax.dot_general(xn, u1_ref[...], _NT,
                                preferred_element_type=jnp.float32)
            + u1b_ref[...])
        h2 = jax.lax.dot_general(h1, u2_ref[...], _NT,
                                 preferred_element_type=jnp.float32) \
            + u2b_ref[...]
        mu2 = jnp.mean(h2, axis=1, keepdims=True)
        var2 = jnp.mean((h2 - mu2) ** 2, axis=1, keepdims=True)
        dmc_ref[...] = (h2 - mu2) * jax.lax.rsqrt(var2 + 1e-5) \
            * now_ref[...] + nob_ref[...]
        lf_sum = (acc_scr[0 * _M:1 * _M, _HD:_HD + 1]
                  + acc_scr[1 * _M:2 * _M, _HD:_HD + 1]
                  + acc_scr[2 * _M:3 * _M, _HD:_HD + 1]
                  + acc_scr[3 * _M:4 * _M, _HD:_HD + 1])
        lf_ref[...] = lf_sum * (1.0 / _H)


def _const(shape):
    return pl.BlockSpec(shape, lambda i: tuple(0 for _ in shape))


@functools.partial(jax.jit, static_argnames=())
def _run(flat, gm, wq, bq_c, p1, p1b_r, p2, p2b_r, mb_c, lnw_r, lnb_r,
         u1, u1b_r, u2, u2b_r, now_r, nob_r):
    nb = _N // _BN
    return pl.pallas_call(
        _body,
        grid=(nb,),
        in_specs=[
            pl.BlockSpec((_BN, _D), lambda i: (i, 0)),
            _const((_M, _CORE)),
            _const((_CORE, _D)),
            _const((_CORE, 1)),
            _const((_D, _CORE)),
            _const((1, _D)),
            _const((_CORE, _D)),
            _const((1, _CORE)),
            _const((_CORE, 1)),
            _const((1, 2 * _CORE)),
            _const((1, 2 * _CORE)),
            _const((2 * _CORE, 2 * _CORE)),
            _const((1, 2 * _CORE)),
            _const((_CORE, 2 * _CORE)),
            _const((1, _CORE)),
            _const((1, _CORE)),
            _const((1, _CORE)),
        ],
        out_specs=[
            pl.BlockSpec((_CORE, _BN), lambda i: (0, i)),
            pl.BlockSpec((_CORE, _BN), lambda i: (0, i)),
            _const((_M, _CORE)),
            _const((_M, 1)),
        ],
        out_shape=[
            jax.ShapeDtypeStruct((_CORE, _N), jnp.float32),
            jax.ShapeDtypeStruct((_CORE, _N), jnp.float32),
            jax.ShapeDtypeStruct((_M, _CORE), jnp.float32),
            jax.ShapeDtypeStruct((_M, 1), jnp.float32),
        ],
        scratch_shapes=[
            pltpu.VMEM((_CORE, _CORE), jnp.float32),
            pltpu.VMEM((_CORE, _HD + 8), jnp.float32),
        ],
        compiler_params=pltpu.CompilerParams(
            dimension_semantics=("arbitrary",)),
    )(flat, gm, wq, bq_c, p1, p1b_r, p2, p2b_r, mb_c, lnw_r, lnb_r,
      u1, u1b_r, u2, u2b_r, now_r, nob_r)


def kernel(batch_queries, global_memory_base, Wq, bq, P1, p1b, P2, p2b,
           mem_bias, ln_w, ln_b, U1, U1b, U2, U2b, no_w, no_b):
    flat = batch_queries.reshape(_N, _D)
    gm = global_memory_base.reshape(_M, _CORE)
    s_all, s2_all, dmc, lf = _run(
        flat, gm, Wq, bq.reshape(_CORE, 1), P1, p1b.reshape(1, _D),
        P2, p2b.reshape(1, _CORE), mem_bias.reshape(_CORE, 1),
        ln_w.reshape(1, 2 * _CORE), ln_b.reshape(1, 2 * _CORE),
        U1, U1b.reshape(1, 2 * _CORE), U2, U2b.reshape(1, _CORE),
        no_w.reshape(1, _CORE), no_b.reshape(1, _CORE))
    return (dmc.reshape(1, _M, _CORE), s_all.reshape(_H, _M, _N),
            s2_all.reshape(_H, _M, _N), lf.reshape(_M))
